# baseline (speedup 1.0000x reference)
import functools

import jax
import jax.numpy as jnp
from jax import lax
from jax.experimental import pallas as pl
from jax.experimental.pallas import tpu as pltpu
from jax.experimental.pallas import tpu_sc as plsc

_DN = (((1,), (1,)), ((), ()))


def _ew_body(x_ref, gw_ref, nw_ref, n_ref, ew_ref):
    xb = x_ref[...]
    clean = jax.lax.dot_general(xb, gw_ref[...], _DN,
                                preferred_element_type=jnp.float32)
    raw = jax.lax.dot_general(xb, nw_ref[...], _DN,
                              preferred_element_type=jnp.float32)
    ew_ref[...] = clean + n_ref[...] * jax.nn.softplus(raw)


def _expert_weights(x, gate_weights, noise_weights, noise):
    n_tokens, d_model = x.shape
    n_experts = gate_weights.shape[0]
    bt = 4096
    return pl.pallas_call(
        _ew_body,
        grid=(n_tokens // bt,),
        in_specs=[
            pl.BlockSpec((bt, d_model), lambda i: (i, 0)),
            pl.BlockSpec((n_experts, d_model), lambda i: (0, 0)),
            pl.BlockSpec((n_experts, d_model), lambda i: (0, 0)),
            pl.BlockSpec((bt, n_experts), lambda i: (i, 0)),
        ],
        out_specs=pl.BlockSpec((bt, n_experts), lambda i: (i, 0)),
        out_shape=jax.ShapeDtypeStruct((n_tokens, n_experts), jnp.float32),
    )(x, gate_weights, noise_weights, noise)


def _sc_route(ew):
    n, e = ew.shape
    info = plsc.get_sparse_core_info()
    n_cores = info.num_cores
    nw = n_cores * info.num_subcores
    rpw = n // nw
    chunk = 256
    mesh = plsc.VectorSubcoreMesh(core_axis_name="c", subcore_axis_name="s")

    @functools.partial(
        pl.kernel,
        out_type=jax.ShapeDtypeStruct((n * e,), jnp.float32),
        mesh=mesh,
        compiler_params=pltpu.CompilerParams(needs_layout_passes=False),
        scratch_types=[
            pltpu.VMEM((chunk * e,), jnp.float32),
            pltpu.VMEM((chunk * e,), jnp.float32),
        ],
    )
    def route(ew_hbm, out_hbm, bin_, bout):
        wid = lax.axis_index("s") * n_cores + lax.axis_index("c")
        base = wid * rpw
        lanes = jnp.arange(16, dtype=jnp.int32)

        def do_chunk(ci, carry):
            elt0 = (base + ci * chunk) * e
            pltpu.sync_copy(ew_hbm.at[pl.ds(elt0, chunk * e)], bin_)

            def do_group(g, carry2):
                eidx0 = (g * 16 + lanes) * e
                neg = jnp.full((16,), -jnp.inf, jnp.float32)
                m1 = neg
                m2 = neg
                for e_i in range(e):
                    v = plsc.load_gather(bin_, [eidx0 + e_i])
                    m2 = jnp.maximum(m2, jnp.minimum(m1, v))
                    m1 = jnp.maximum(m1, v)
                e2 = jnp.exp(m2 - m1)
                inv = 1.0 / (1.0 + e2)
                p2v = e2 * inv
                zero = jnp.zeros((16,), jnp.float32)
                for e_i in range(e):
                    v = plsc.load_gather(bin_, [eidx0 + e_i])
                    eq1 = v == m1
                    eq2 = (v == m2) & jnp.logical_not(eq1)
                    outv = jnp.where(eq1, inv, jnp.where(eq2, p2v, zero))
                    plsc.store_scatter(bout, [eidx0 + e_i], outv)
                return carry2

            lax.fori_loop(0, chunk // 16, do_group, 0)
            pltpu.sync_copy(bout, out_hbm.at[pl.ds(elt0, chunk * e)])
            return carry

        lax.fori_loop(0, rpw // chunk, do_chunk, 0)

    return route(ew.reshape(-1)).reshape(n, e)


def kernel(x, gate_weights, noise_weights, noise):
    ew = _expert_weights(x, gate_weights, noise_weights, noise)
    return _sc_route(ew)


# speedup vs baseline: 2.8394x; 2.8394x over previous
import jax
import jax.numpy as jnp
from jax.experimental import pallas as pl

_DN = (((1,), (1,)), ((), ()))


def _router_body(x_ref, gw_ref, nw_ref, n_ref, o_ref):
    xb = x_ref[...]
    clean = jax.lax.dot_general(xb, gw_ref[...], _DN,
                                preferred_element_type=jnp.float32)
    raw = jax.lax.dot_general(xb, nw_ref[...], _DN,
                              preferred_element_type=jnp.float32)
    ns = jax.nn.softplus(raw)
    ew = clean + n_ref[...] * ns
    m1 = jnp.max(ew, axis=1, keepdims=True)
    is1 = ew == m1
    ew2 = jnp.where(is1, -jnp.inf, ew)
    m2 = jnp.max(ew2, axis=1, keepdims=True)
    is2 = ew2 == m2
    e2 = jnp.exp(m2 - m1)
    inv = 1.0 / (1.0 + e2)
    o_ref[...] = jnp.where(is1, inv, jnp.where(is2, e2 * inv, 0.0))


def kernel(x, gate_weights, noise_weights, noise):
    n_tokens, d_model = x.shape
    n_experts = gate_weights.shape[0]
    bt = 4096
    return pl.pallas_call(
        _router_body,
        grid=(n_tokens // bt,),
        in_specs=[
            pl.BlockSpec((bt, d_model), lambda i: (i, 0)),
            pl.BlockSpec((n_experts, d_model), lambda i: (0, 0)),
            pl.BlockSpec((n_experts, d_model), lambda i: (0, 0)),
            pl.BlockSpec((bt, n_experts), lambda i: (i, 0)),
        ],
        out_specs=pl.BlockSpec((bt, n_experts), lambda i: (i, 0)),
        out_shape=jax.ShapeDtypeStruct((n_tokens, n_experts), jnp.float32),
    )(x, gate_weights, noise_weights, noise)
